# baseline (speedup 1.0000x reference)
import functools

import jax
import jax.numpy as jnp
from jax import lax
from jax.experimental import pallas as pl
from jax.experimental.pallas import tpu as pltpu
from jax.experimental.pallas import tpu_sc as plsc

B, J, K, H, V = 16, 4, 2048, 256, 65536
PAIRS = B * J
L = 16
NC, NS = 2, 16
NW = NC * NS
PPW = PAIRS // NW
VBLK = 2048
NSTREAM = 4


def _tc_scores(stims, embed):
    def mm(stims_ref, *rest):
        emb_refs, out_ref = rest[:NSTREAM], rest[NSTREAM]
        s = stims_ref[...]
        outs = [
            lax.dot_general(
                s, e[...],
                dimension_numbers=(((1,), (1,)), ((), ())),
                preferred_element_type=jnp.float32,
                precision=lax.Precision.HIGHEST,
            )
            for e in emb_refs
        ]
        out_ref[...] = jnp.concatenate(outs, axis=1)

    emb_specs = [
        pl.BlockSpec((VBLK, H), functools.partial(
            lambda i, j: (NSTREAM * i + j, 0), j=j))
        for j in range(NSTREAM)
    ]
    return pl.pallas_call(
        mm,
        grid=(V // (NSTREAM * VBLK),),
        in_specs=[pl.BlockSpec((B, H), lambda i: (0, 0))] + emb_specs,
        out_specs=pl.BlockSpec((B, NSTREAM * VBLK), lambda i: (0, i)),
        out_shape=jax.ShapeDtypeStruct((B, V), jnp.float32),
    )(stims, *([embed] * NSTREAM))


def _sc_gather_argmax(scores, idx, lens):
    mesh = plsc.VectorSubcoreMesh(core_axis_name="c", subcore_axis_name="s")

    @functools.partial(
        pl.kernel,
        mesh=mesh,
        compiler_params=pltpu.CompilerParams(needs_layout_passes=False),
        out_type=[
            jax.ShapeDtypeStruct((PAIRS, K), jnp.float32),
            jax.ShapeDtypeStruct((PAIRS, L), jnp.int32),
        ],
        scratch_types=[
            pltpu.VMEM((V,), jnp.float32),
            pltpu.VMEM((K,), jnp.int32),
            pltpu.VMEM((K,), jnp.float32),
            pltpu.VMEM((L,), jnp.int32),
            pltpu.VMEM((PAIRS,), jnp.int32),
        ],
    )
    def k(scores_hbm, idx_hbm, lens_hbm, x_hbm, xidx_hbm,
          row_v, idx_v, xbuf_v, xidx_v, lens_v):
        wid = lax.axis_index("s") * NC + lax.axis_index("c")
        b = wid // (NW // B)
        pltpu.sync_copy(scores_hbm.at[b], row_v)
        pltpu.sync_copy(lens_hbm, lens_v)
        lane = lax.broadcasted_iota(jnp.int32, (L,), 0)
        neg = jnp.full((L,), -1e9, jnp.float32)
        for jj in range(PPW):
            p = wid * PPW + jj
            pltpu.sync_copy(idx_hbm.at[p], idx_v)
            ln = plsc.load_gather(lens_v, [jnp.full((L,), p, jnp.int32)])

            def body(i, carry, ln=ln):
                best_val, best_idx = carry
                idxv = idx_v[pl.ds(i * L, L)]
                vals = plsc.load_gather(row_v, [idxv])
                xbuf_v[pl.ds(i * L, L)] = vals
                kv = lane + i * L
                mval = jnp.where(kv < ln, vals, neg)
                upd = mval > best_val
                return (jnp.where(upd, mval, best_val),
                        jnp.where(upd, kv, best_idx))

            bv0 = jnp.full((L,), -jnp.inf, jnp.float32)
            bi0 = jnp.zeros((L,), jnp.int32)
            bv, bi = lax.fori_loop(0, K // L, body, (bv0, bi0))
            mx = jnp.max(bv, axis=0)
            cand = jnp.where(bv == mx, bi, jnp.int32(K))
            amin = jnp.min(cand, axis=0)
            xidx_v[...] = jnp.full((L,), amin, jnp.int32)
            pltpu.sync_copy(xbuf_v, x_hbm.at[p])
            pltpu.sync_copy(xidx_v, xidx_hbm.at[p])

    return k(scores, idx, lens)


def kernel(stims, embed, atn_idx, lens):
    scores = _tc_scores(stims, embed)
    idx = atn_idx.reshape(PAIRS, K).astype(jnp.int32)
    lens_flat = lens.reshape(PAIRS).astype(jnp.int32)
    x_flat, xidx = _sc_gather_argmax(scores, idx, lens_flat)
    x = x_flat.reshape(B, J, K)
    xIdx = xidx[:, 0].reshape(B, J)
    return (x, xIdx)


# speedup vs baseline: 1.0140x; 1.0140x over previous
import functools

import jax
import jax.numpy as jnp
from jax import lax
from jax.experimental import pallas as pl
from jax.experimental.pallas import tpu as pltpu
from jax.experimental.pallas import tpu_sc as plsc

B, J, K, H, V = 16, 4, 2048, 256, 65536
PAIRS = B * J
L = 16
NC, NS = 2, 16
NW = NC * NS
PPW = PAIRS // NW
NPHASE = 2
VH = V // NPHASE
VBLK = 4096


def _tc_scores_phase(stims, embed, ph):
    def mm(stims_ref, emb_ref, out_ref):
        out_ref[...] = lax.dot_general(
            stims_ref[...], emb_ref[...],
            dimension_numbers=(((1,), (1,)), ((), ())),
            preferred_element_type=jnp.float32,
            precision=lax.Precision.HIGHEST,
        )

    base_blk = ph * (VH // VBLK)
    return pl.pallas_call(
        mm,
        grid=(VH // VBLK,),
        in_specs=[
            pl.BlockSpec((B, H), lambda i: (0, 0)),
            pl.BlockSpec((VBLK, H), lambda i: (base_blk + i, 0)),
        ],
        out_specs=pl.BlockSpec((B, VBLK), lambda i: (0, i)),
        out_shape=jax.ShapeDtypeStruct((B, VH), jnp.float32),
    )(stims, embed)


def _sc_phase(scores_ph, atn_idx, lens, prev_x, ph):
    last = ph == NPHASE - 1
    base = ph * VH
    mesh = plsc.VectorSubcoreMesh(core_axis_name="c", subcore_axis_name="s")

    out_type = [jax.ShapeDtypeStruct((B, J, K), jnp.float32)]
    scratch = [
        pltpu.VMEM((VH,), jnp.float32),
        pltpu.VMEM((K,), jnp.int32),
        pltpu.VMEM((K,), jnp.float32),
        pltpu.VMEM((K,), jnp.float32),
    ]
    if last:
        out_type.append(jax.ShapeDtypeStruct((PAIRS, L), jnp.int32))
        scratch.append(pltpu.VMEM((L,), jnp.int32))
        scratch.append(pltpu.VMEM((PAIRS,), jnp.int32))

    @functools.partial(
        pl.kernel,
        mesh=mesh,
        compiler_params=pltpu.CompilerParams(needs_layout_passes=False),
        out_type=out_type,
        scratch_types=scratch,
    )
    def k(scores_hbm, idx_hbm, *rest):
        prev_hbm = None
        if ph > 0:
            prev_hbm, *rest = rest
        if last:
            lens_hbm, x_hbm, xidx_hbm, row_v, idx_v, xbuf_v, prev_v, \
                xidx_v, lens_v = rest
        else:
            x_hbm, row_v, idx_v, xbuf_v, prev_v = rest
        wid = lax.axis_index("s") * NC + lax.axis_index("c")
        b = wid // (NW // B)
        pltpu.sync_copy(scores_hbm.at[b], row_v)
        if last:
            pltpu.sync_copy(lens_hbm, lens_v)
        lane = lax.broadcasted_iota(jnp.int32, (L,), 0)
        neg = jnp.full((L,), -1e9, jnp.float32)
        zero = jnp.zeros((L,), jnp.int32)
        for jj in range(PPW):
            p = wid * PPW + jj
            j = p % J
            pltpu.sync_copy(idx_hbm.at[b, j], idx_v)
            if ph > 0:
                pltpu.sync_copy(prev_hbm.at[b, j], prev_v)
            if last:
                ln = plsc.load_gather(
                    lens_v, [jnp.full((L,), p, jnp.int32)])

            def body(i, carry):
                best_val, best_idx = carry
                idxv = idx_v[pl.ds(i * L, L)]
                local = idxv - base
                inrange = (idxv >= base) & (local < VH)
                safe = jnp.where(inrange, local, zero)
                vals = plsc.load_gather(row_v, [safe])
                if ph > 0:
                    prev = prev_v[pl.ds(i * L, L)]
                else:
                    prev = jnp.zeros((L,), jnp.float32)
                merged = jnp.where(inrange, vals, prev)
                xbuf_v[pl.ds(i * L, L)] = merged
                if last:
                    kv = lane + i * L
                    mval = jnp.where(kv < ln, merged, neg)
                    upd = mval > best_val
                    return (jnp.where(upd, mval, best_val),
                            jnp.where(upd, kv, best_idx))
                return carry

            bv0 = jnp.full((L,), -jnp.inf, jnp.float32)
            bi0 = jnp.zeros((L,), jnp.int32)
            bv, bi = lax.fori_loop(0, K // L, body, (bv0, bi0))
            if last:
                mx = jnp.max(bv, axis=0)
                cand = jnp.where(bv == mx, bi, jnp.int32(K))
                amin = jnp.min(cand, axis=0)
                xidx_v[...] = jnp.full((L,), amin, jnp.int32)
                pltpu.sync_copy(xidx_v, xidx_hbm.at[p])
            pltpu.sync_copy(xbuf_v, x_hbm.at[b, j])

    args = [scores_ph, atn_idx]
    if ph > 0:
        args.append(prev_x)
    if last:
        args.append(lens)
    return k(*args)


def kernel(stims, embed, atn_idx, lens):
    idx = atn_idx.astype(jnp.int32)
    lens_flat = lens.reshape(PAIRS).astype(jnp.int32)
    x = None
    for ph in range(NPHASE - 1):
        scores_ph = _tc_scores_phase(stims, embed, ph)
        (x,) = _sc_phase(scores_ph, idx, lens_flat, x, ph)
    scores_ph = _tc_scores_phase(stims, embed, NPHASE - 1)
    x, xidx = _sc_phase(scores_ph, idx, lens_flat, x, NPHASE - 1)
    xIdx = xidx[:, 0].reshape(B, J)
    return (x, xIdx)


# speedup vs baseline: 1.0748x; 1.0600x over previous
import functools

import jax
import jax.numpy as jnp
from jax import lax
from jax.experimental import pallas as pl
from jax.experimental.pallas import tpu as pltpu
from jax.experimental.pallas import tpu_sc as plsc

B, J, K, H, V = 16, 4, 2048, 256, 65536
PAIRS = B * J
L = 16
NC, NS = 2, 16
NW = NC * NS
PPW = PAIRS // NW
NPHASE = 2
VH = V // NPHASE
VBLK = 4096


def _tc_scores_phase(stims, embed, ph):
    def mm(stims_ref, emb_ref, out_ref):
        out_ref[...] = lax.dot_general(
            stims_ref[...], emb_ref[...],
            dimension_numbers=(((1,), (1,)), ((), ())),
            preferred_element_type=jnp.float32,
            precision=lax.Precision.HIGHEST,
        )

    base_blk = ph * (VH // VBLK)
    return pl.pallas_call(
        mm,
        grid=(VH // VBLK,),
        in_specs=[
            pl.BlockSpec((B, H), lambda i: (0, 0)),
            pl.BlockSpec((VBLK, H), lambda i: (base_blk + i, 0)),
        ],
        out_specs=pl.BlockSpec((B, VBLK), lambda i: (0, i)),
        out_shape=jax.ShapeDtypeStruct((B, VH), jnp.float32),
    )(stims, embed)


def _sc_phase(scores_ph, atn_idx, lens, prev_x, ph):
    last = ph == NPHASE - 1
    base = ph * VH
    mesh = plsc.VectorSubcoreMesh(core_axis_name="c", subcore_axis_name="s")

    out_type = [jax.ShapeDtypeStruct((B, J, K), jnp.float32)]
    scratch = [
        pltpu.VMEM((VH,), jnp.float32),
        pltpu.VMEM((PPW, K), jnp.int32),
        pltpu.VMEM((PPW, K), jnp.float32),
        pltpu.VMEM((PPW, K), jnp.float32),
        pltpu.SemaphoreType.DMA,
        pltpu.SemaphoreType.DMA,
    ]
    if last:
        out_type.append(jax.ShapeDtypeStruct((PAIRS, L), jnp.int32))
        scratch.append(pltpu.VMEM((PPW, L), jnp.int32))
        scratch.append(pltpu.VMEM((PAIRS,), jnp.int32))

    @functools.partial(
        pl.kernel,
        mesh=mesh,
        compiler_params=pltpu.CompilerParams(needs_layout_passes=False),
        out_type=out_type,
        scratch_types=scratch,
    )
    def k(scores_hbm, idx_hbm, *rest):
        prev_hbm = None
        if ph > 0:
            prev_hbm, *rest = rest
        if last:
            lens_hbm, x_hbm, xidx_hbm, row_v, idx_v, xbuf_v, prev_v, \
                sem_in, sem_out, xidx_v, lens_v = rest
        else:
            x_hbm, row_v, idx_v, xbuf_v, prev_v, sem_in, sem_out = rest
        wid = lax.axis_index("s") * NC + lax.axis_index("c")
        b = wid // (NW // B)
        copies = [pltpu.async_copy(scores_hbm.at[b], row_v, sem_in)]
        for jj in range(PPW):
            j = (wid * PPW + jj) % J
            copies.append(
                pltpu.async_copy(idx_hbm.at[b, j], idx_v.at[jj], sem_in))
            if ph > 0:
                copies.append(
                    pltpu.async_copy(prev_hbm.at[b, j], prev_v.at[jj],
                                     sem_in))
        if last:
            copies.append(pltpu.async_copy(lens_hbm, lens_v, sem_in))
        for c in copies:
            c.wait()
        lane = lax.broadcasted_iota(jnp.int32, (L,), 0)
        neg = jnp.full((L,), -1e9, jnp.float32)
        zero = jnp.zeros((L,), jnp.int32)
        writes = []
        for jj in range(PPW):
            p = wid * PPW + jj
            j = p % J
            if last:
                ln = plsc.load_gather(
                    lens_v, [jnp.full((L,), p, jnp.int32)])

            def body(i, carry, jj=jj):
                best_val, best_idx = carry
                idxv = idx_v[jj, pl.ds(i * L, L)]
                local = idxv - base
                inrange = (idxv >= base) & (local < VH)
                safe = jnp.where(inrange, local, zero)
                vals = plsc.load_gather(row_v, [safe])
                if ph > 0:
                    prev = prev_v[jj, pl.ds(i * L, L)]
                else:
                    prev = jnp.zeros((L,), jnp.float32)
                merged = jnp.where(inrange, vals, prev)
                xbuf_v[jj, pl.ds(i * L, L)] = merged
                if last:
                    kv = lane + i * L
                    mval = jnp.where(kv < ln, merged, neg)
                    upd = mval > best_val
                    return (jnp.where(upd, mval, best_val),
                            jnp.where(upd, kv, best_idx))
                return carry

            bv0 = jnp.full((L,), -jnp.inf, jnp.float32)
            bi0 = jnp.zeros((L,), jnp.int32)
            bv, bi = lax.fori_loop(0, K // L, body, (bv0, bi0))
            if last:
                mx = jnp.max(bv, axis=0)
                cand = jnp.where(bv == mx, bi, jnp.int32(K))
                amin = jnp.min(cand, axis=0)
                xidx_v[jj] = jnp.full((L,), amin, jnp.int32)
                writes.append(
                    pltpu.async_copy(xidx_v.at[jj], xidx_hbm.at[p],
                                     sem_out))
            writes.append(
                pltpu.async_copy(xbuf_v.at[jj], x_hbm.at[b, j], sem_out))
        for w in writes:
            w.wait()

    args = [scores_ph, atn_idx]
    if ph > 0:
        args.append(prev_x)
    if last:
        args.append(lens)
    return k(*args)


def kernel(stims, embed, atn_idx, lens):
    idx = atn_idx.astype(jnp.int32)
    lens_flat = lens.reshape(PAIRS).astype(jnp.int32)
    x = None
    for ph in range(NPHASE - 1):
        scores_ph = _tc_scores_phase(stims, embed, ph)
        (x,) = _sc_phase(scores_ph, idx, lens_flat, x, ph)
    scores_ph = _tc_scores_phase(stims, embed, NPHASE - 1)
    x, xidx = _sc_phase(scores_ph, idx, lens_flat, x, NPHASE - 1)
    xIdx = xidx[:, 0].reshape(B, J)
    return (x, xIdx)
